# baseline (speedup 1.0000x reference)
import functools

import jax
import jax.numpy as jnp
from jax import lax
from jax.experimental import pallas as pl
from jax.experimental.pallas import tpu as pltpu
from jax.experimental.pallas import tpu_sc as plsc

B, C, H, W, COUT, K = 8, 384, 24, 24, 768, 9
N = H * W
BN = B * N
NKB = float(B * N * K)

NC, NS = 2, 16
NW = NC * NS
PW = BN // NW
CH = 8
NCH = PW // CH
ROWS = CH * K
LG = COUT // 16


def _prep_body(x_ref, wd_ref, w2_ref, idx_ref, a_ref, bm_ref, stat_ref):
    b = pl.program_id(0)
    x = x_ref[0]
    nsq = jnp.sum(x * x, axis=0, keepdims=True)
    xn = x / jnp.maximum(jnp.sqrt(nsq), 1e-12)
    sqv = jnp.sum(xn * xn, axis=0, keepdims=True)
    gram = lax.dot_general(xn, xn, (((0,), (0,)), ((), ())),
                           preferred_element_type=jnp.float32)
    rank = sqv - 2.0 * gram
    iota = lax.broadcasted_iota(jnp.int32, (N, N), 1)
    rows = []
    d = rank
    e = jnp.zeros((N, N), jnp.float32)
    for _ in range(K):
        m = jnp.min(d, axis=1, keepdims=True)
        cand = jnp.where(d == m, iota, N)
        am = jnp.min(cand, axis=1, keepdims=True)
        rows.append(am)
        hit = iota == am
        d = jnp.where(hit, jnp.inf, d)
        e = e + jnp.where(hit, 1.0, 0.0)
    idx_mat = jnp.concatenate(rows, axis=1) + b * N
    pad = jnp.zeros((N, 16 - K), jnp.int32) + b * N
    idx_ref[0] = jnp.concatenate([idx_mat, pad], axis=1)

    a = lax.dot_general(x, wd_ref[...], (((0,), (0,)), ((), ())),
                        preferred_element_type=jnp.float32)
    bm = lax.dot_general(x, w2_ref[...], (((0,), (0,)), ((), ())),
                         preferred_element_type=jnp.float32)
    a_ref[0] = a
    bm_ref[0] = bm
    s = lax.dot_general(e, bm, (((1,), (0,)), ((), ())),
                        preferred_element_type=jnp.float32)
    cnt = jnp.sum(e, axis=0, keepdims=True)
    bg2 = lax.dot_general(cnt, bm * bm, (((1,), (0,)), ((), ())),
                          preferred_element_type=jnp.float32)
    stat_ref[0] = jnp.concatenate([
        jnp.sum(a, axis=0, keepdims=True),
        jnp.sum(a * a, axis=0, keepdims=True),
        jnp.sum(s, axis=0, keepdims=True),
        bg2,
        jnp.sum(a * s, axis=0, keepdims=True),
    ], axis=0)


@functools.cache
def _prep(nb):
    return pl.pallas_call(
        _prep_body,
        grid=(nb,),
        in_specs=[
            pl.BlockSpec((1, C, N), lambda b: (b, 0, 0)),
            pl.BlockSpec((C, COUT), lambda b: (0, 0)),
            pl.BlockSpec((C, COUT), lambda b: (0, 0)),
        ],
        out_specs=[
            pl.BlockSpec((1, N, 16), lambda b: (b, 0, 0)),
            pl.BlockSpec((1, N, COUT), lambda b: (b, 0, 0)),
            pl.BlockSpec((1, N, COUT), lambda b: (b, 0, 0)),
            pl.BlockSpec((1, 5, COUT), lambda b: (b, 0, 0)),
        ],
        out_shape=[
            jax.ShapeDtypeStruct((nb, N, 16), jnp.int32),
            jax.ShapeDtypeStruct((nb, N, COUT), jnp.float32),
            jax.ShapeDtypeStruct((nb, N, COUT), jnp.float32),
            jax.ShapeDtypeStruct((nb, 5, COUT), jnp.float32),
        ],
    )


def _sc_body(pw, bm_hbm, idx_hbm, m_hbm, mn_hbm,
             idx_v, rows0, rows1, m_v, mn_v, gsem0, gsem1, ssem):
    nch = pw // CH
    cid = lax.axis_index("c")
    sid = lax.axis_index("s")
    wid = sid * NC + cid
    base_pt = wid * pw

    pltpu.sync_copy(idx_hbm.at[pl.ds(base_pt * K, pw * K)], idx_v)

    rows_bufs = (rows0, rows1)
    gsems = (gsem0, gsem1)

    def _start_gather(t):
        buf = rows_bufs[t % 2]
        return pltpu.async_copy(
            bm_hbm.at[idx_v.at[pl.ds(t * ROWS, ROWS)]], buf, gsems[t % 2])

    def _compute(t):
        rows_v = rows_bufs[t % 2]

        def _pt_body(p, _):
            def _cg_body(cg, _):
                for u in range(3):
                    col = pl.ds(cg * 48 + u * 16, 16)
                    r = rows_v[p * K, col]
                    mx = r
                    mn = r
                    for k in range(1, K):
                        v = rows_v[p * K + k, col]
                        mx = jnp.maximum(mx, v)
                        mn = jnp.minimum(mn, v)
                    m_v[p, col] = mx
                    mn_v[p, col] = mn
                return 0

            lax.fori_loop(0, LG // 3, _cg_body, 0)
            return 0

        lax.fori_loop(0, CH, _pt_body, 0)

    pending = _start_gather(0)
    store_m = None
    for t in range(nch):
        nxt = _start_gather(t + 1) if t + 1 < nch else None
        pending.wait()
        if store_m is not None:
            store_m.wait()
            store_mn.wait()
        _compute(t)
        p0 = base_pt + t * CH
        store_m = pltpu.async_copy(m_v, m_hbm.at[pl.ds(p0, CH)], ssem)
        store_mn = pltpu.async_copy(mn_v, mn_hbm.at[pl.ds(p0, CH)], ssem)
        pending = nxt
    store_m.wait()
    store_mn.wait()


@functools.cache
def _sc_gather(npts):
    pw = npts // NW
    return pl.kernel(
        functools.partial(_sc_body, pw),
        out_type=[
            jax.ShapeDtypeStruct((npts, COUT), jnp.float32),
            jax.ShapeDtypeStruct((npts, COUT), jnp.float32),
        ],
        mesh=plsc.VectorSubcoreMesh(core_axis_name="c", subcore_axis_name="s",
                                    num_cores=NC, num_subcores=NS),
        scratch_types=[
            pltpu.VMEM((pw * K,), jnp.int32),
            pltpu.VMEM((ROWS, COUT), jnp.float32),
            pltpu.VMEM((ROWS, COUT), jnp.float32),
            pltpu.VMEM((CH, COUT), jnp.float32),
            pltpu.VMEM((CH, COUT), jnp.float32),
            pltpu.SemaphoreType.DMA,
            pltpu.SemaphoreType.DMA,
            pltpu.SemaphoreType.DMA,
        ],
    )


def _gelu(y):
    return 0.5 * y * (1.0 + lax.erf(y * 0.7071067811865476))


def _fin_body(a_ref, m_ref, mn_ref, stat_ref, bnw_ref, bnb_ref, o_ref):
    sum_a = jnp.sum(stat_ref[:, 0, :], axis=0, keepdims=True)
    sum_a2 = jnp.sum(stat_ref[:, 1, :], axis=0, keepdims=True)
    sum_bg = jnp.sum(stat_ref[:, 2, :], axis=0, keepdims=True)
    sum_bg2 = jnp.sum(stat_ref[:, 3, :], axis=0, keepdims=True)
    sum_cr = jnp.sum(stat_ref[:, 4, :], axis=0, keepdims=True)
    mean = (K * sum_a + sum_bg) / NKB
    ex2 = (K * sum_a2 + 2.0 * sum_cr + sum_bg2) / NKB
    var = ex2 - mean * mean
    inv = bnw_ref[...] * lax.rsqrt(var + 1e-5)
    a = a_ref[0]
    y1 = (a + m_ref[0] - mean) * inv + bnb_ref[...]
    y2 = (a + mn_ref[0] - mean) * inv + bnb_ref[...]
    o_ref[0] = jnp.maximum(_gelu(y1), _gelu(y2))


@functools.cache
def _finalize(nb):
    return pl.pallas_call(
        _fin_body,
        grid=(nb,),
        in_specs=[
            pl.BlockSpec((1, N, COUT), lambda b: (b, 0, 0)),
            pl.BlockSpec((1, N, COUT), lambda b: (b, 0, 0)),
            pl.BlockSpec((1, N, COUT), lambda b: (b, 0, 0)),
            pl.BlockSpec((B, 5, COUT), lambda b: (0, 0, 0)),
            pl.BlockSpec((1, COUT), lambda b: (0, 0)),
            pl.BlockSpec((1, COUT), lambda b: (0, 0)),
        ],
        out_specs=[pl.BlockSpec((1, N, COUT), lambda b: (b, 0, 0))],
        out_shape=[jax.ShapeDtypeStruct((nb, N, COUT), jnp.float32)],
    )


def kernel(x, W_conv, b_conv, bn_w, bn_b):
    del b_conv
    xr = x.reshape(B, C, N)
    w1 = W_conv[:, :C]
    w2 = W_conv[:, C:]
    wd = (w1 - w2).T
    w2t = w2.T
    hb = B // 2
    hn = hb * N
    bnw = bn_w.reshape(1, COUT)
    bnb = bn_b.reshape(1, COUT)
    halves = []
    for h in range(2):
        xh = xr[h * hb:(h + 1) * hb]
        idx, a, bm, stat = _prep(hb)(xh, wd, w2t)
        idx_flat = idx[:, :, :K].reshape(hn * K)
        m, mn = _sc_gather(hn)(bm.reshape(hn, COUT), idx_flat)
        halves.append((a, m, mn, stat))
    stat_all = jnp.concatenate([halves[0][3], halves[1][3]], axis=0)
    outs = []
    for h in range(2):
        a, m, mn, _ = halves[h]
        (o,) = _finalize(hb)(a, m.reshape(hb, N, COUT),
                             mn.reshape(hb, N, COUT), stat_all, bnw, bnb)
        outs.append(o)
    out = jnp.concatenate(outs, axis=0)
    return out.transpose(0, 2, 1).reshape(B, COUT, H, W)


# speedup vs baseline: 1.1024x; 1.1024x over previous
import functools

import jax
import jax.numpy as jnp
from jax import lax
from jax.experimental import pallas as pl
from jax.experimental.pallas import tpu as pltpu
from jax.experimental.pallas import tpu_sc as plsc

B, C, H, W, COUT, K = 8, 384, 24, 24, 768, 9
N = H * W
BN = B * N
NKB = float(B * N * K)

NC, NS = 2, 16
NW = NC * NS
PW = BN // NW
CH = 8
NCH = PW // CH
ROWS = CH * K
LG = COUT // 16


def _prep_body(x_ref, wd_ref, w2_ref, idx_ref, a_ref, bm_ref, stat_ref):
    b = pl.program_id(0)
    x = x_ref[0]
    nsq = jnp.sum(x * x, axis=0, keepdims=True)
    xn = x / jnp.maximum(jnp.sqrt(nsq), 1e-12)
    sqv = jnp.sum(xn * xn, axis=0, keepdims=True)
    gram = lax.dot_general(xn, xn, (((0,), (0,)), ((), ())),
                           preferred_element_type=jnp.float32)
    rank = sqv - 2.0 * gram
    iota = lax.broadcasted_iota(jnp.int32, (N, N), 1)
    rows = []
    d = rank
    e = jnp.zeros((N, N), jnp.float32)
    for _ in range(K):
        m = jnp.min(d, axis=1, keepdims=True)
        cand = jnp.where(d == m, iota, N)
        am = jnp.min(cand, axis=1, keepdims=True)
        rows.append(am)
        hit = iota == am
        d = jnp.where(hit, jnp.inf, d)
        e = e + jnp.where(hit, 1.0, 0.0)
    idx_mat = jnp.concatenate(rows, axis=1) + b * N
    pad = jnp.zeros((N, 16 - K), jnp.int32) + b * N
    idx_ref[0] = jnp.concatenate([idx_mat, pad], axis=1)

    a = lax.dot_general(x, wd_ref[...], (((0,), (0,)), ((), ())),
                        preferred_element_type=jnp.float32)
    bm = lax.dot_general(x, w2_ref[...], (((0,), (0,)), ((), ())),
                         preferred_element_type=jnp.float32)
    a_ref[0] = a
    bm_ref[0] = bm
    s = lax.dot_general(e, bm, (((1,), (0,)), ((), ())),
                        preferred_element_type=jnp.float32)
    cnt = jnp.sum(e, axis=0, keepdims=True)
    bg2 = lax.dot_general(cnt, bm * bm, (((1,), (0,)), ((), ())),
                          preferred_element_type=jnp.float32)
    stat_ref[0] = jnp.concatenate([
        jnp.sum(a, axis=0, keepdims=True),
        jnp.sum(a * a, axis=0, keepdims=True),
        jnp.sum(s, axis=0, keepdims=True),
        bg2,
        jnp.sum(a * s, axis=0, keepdims=True),
    ], axis=0)


@functools.cache
def _prep(nb):
    return pl.pallas_call(
        _prep_body,
        grid=(nb,),
        in_specs=[
            pl.BlockSpec((1, C, N), lambda b: (b, 0, 0)),
            pl.BlockSpec((C, COUT), lambda b: (0, 0)),
            pl.BlockSpec((C, COUT), lambda b: (0, 0)),
        ],
        out_specs=[
            pl.BlockSpec((1, N, 16), lambda b: (b, 0, 0)),
            pl.BlockSpec((1, N, COUT), lambda b: (b, 0, 0)),
            pl.BlockSpec((1, N, COUT), lambda b: (b, 0, 0)),
            pl.BlockSpec((1, 5, COUT), lambda b: (b, 0, 0)),
        ],
        out_shape=[
            jax.ShapeDtypeStruct((nb, N, 16), jnp.int32),
            jax.ShapeDtypeStruct((nb, N, COUT), jnp.float32),
            jax.ShapeDtypeStruct((nb, N, COUT), jnp.float32),
            jax.ShapeDtypeStruct((nb, 5, COUT), jnp.float32),
        ],
    )


def _sc_body(pw, bm_hbm, idx_hbm, m_hbm, mn_hbm,
             idx_v, rows0, rows1, m_v, mn_v, gsem0, gsem1, ssem):
    nch = pw // CH
    cid = lax.axis_index("c")
    sid = lax.axis_index("s")
    wid = sid * NC + cid
    base_pt = wid * pw

    pltpu.sync_copy(idx_hbm.at[pl.ds(base_pt * K, pw * K)], idx_v)

    rows_bufs = (rows0, rows1)
    gsems = (gsem0, gsem1)

    def _start_gather(t):
        buf = rows_bufs[t % 2]
        return pltpu.async_copy(
            bm_hbm.at[idx_v.at[pl.ds(t * ROWS, ROWS)]], buf, gsems[t % 2])

    def _compute(t):
        rows_v = rows_bufs[t % 2]

        def _cg_body(cg, _):
            col = pl.ds(cg * 16, 16)
            for p in range(CH):
                r = rows_v[p * K, col]
                mx = r
                mn = r
                for k in range(1, K):
                    v = rows_v[p * K + k, col]
                    mx = jnp.maximum(mx, v)
                    mn = jnp.minimum(mn, v)
                m_v[p, col] = mx
                mn_v[p, col] = mn
            return 0

        lax.fori_loop(0, LG, _cg_body, 0)

    pending = _start_gather(0)
    store_m = None
    for t in range(nch):
        nxt = _start_gather(t + 1) if t + 1 < nch else None
        pending.wait()
        if store_m is not None:
            store_m.wait()
            store_mn.wait()
        _compute(t)
        p0 = base_pt + t * CH
        store_m = pltpu.async_copy(m_v, m_hbm.at[pl.ds(p0, CH)], ssem)
        store_mn = pltpu.async_copy(mn_v, mn_hbm.at[pl.ds(p0, CH)], ssem)
        pending = nxt
    store_m.wait()
    store_mn.wait()


@functools.cache
def _sc_gather(npts):
    pw = npts // NW
    return pl.kernel(
        functools.partial(_sc_body, pw),
        out_type=[
            jax.ShapeDtypeStruct((npts, COUT), jnp.float32),
            jax.ShapeDtypeStruct((npts, COUT), jnp.float32),
        ],
        mesh=plsc.VectorSubcoreMesh(core_axis_name="c", subcore_axis_name="s",
                                    num_cores=NC, num_subcores=NS),
        scratch_types=[
            pltpu.VMEM((pw * K,), jnp.int32),
            pltpu.VMEM((ROWS, COUT), jnp.float32),
            pltpu.VMEM((ROWS, COUT), jnp.float32),
            pltpu.VMEM((CH, COUT), jnp.float32),
            pltpu.VMEM((CH, COUT), jnp.float32),
            pltpu.SemaphoreType.DMA,
            pltpu.SemaphoreType.DMA,
            pltpu.SemaphoreType.DMA,
        ],
    )


def _gelu(y):
    return 0.5 * y * (1.0 + lax.erf(y * 0.7071067811865476))


def _fin_body(a_ref, m_ref, mn_ref, stat_ref, bnw_ref, bnb_ref, o_ref):
    sum_a = jnp.sum(stat_ref[:, 0, :], axis=0, keepdims=True)
    sum_a2 = jnp.sum(stat_ref[:, 1, :], axis=0, keepdims=True)
    sum_bg = jnp.sum(stat_ref[:, 2, :], axis=0, keepdims=True)
    sum_bg2 = jnp.sum(stat_ref[:, 3, :], axis=0, keepdims=True)
    sum_cr = jnp.sum(stat_ref[:, 4, :], axis=0, keepdims=True)
    mean = (K * sum_a + sum_bg) / NKB
    ex2 = (K * sum_a2 + 2.0 * sum_cr + sum_bg2) / NKB
    var = ex2 - mean * mean
    inv = bnw_ref[...] * lax.rsqrt(var + 1e-5)
    a = a_ref[0]
    y1 = (a + m_ref[0] - mean) * inv + bnb_ref[...]
    y2 = (a + mn_ref[0] - mean) * inv + bnb_ref[...]
    o_ref[0] = jnp.maximum(_gelu(y1), _gelu(y2))


@functools.cache
def _finalize(nb):
    return pl.pallas_call(
        _fin_body,
        grid=(nb,),
        in_specs=[
            pl.BlockSpec((1, N, COUT), lambda b: (b, 0, 0)),
            pl.BlockSpec((1, N, COUT), lambda b: (b, 0, 0)),
            pl.BlockSpec((1, N, COUT), lambda b: (b, 0, 0)),
            pl.BlockSpec((B, 5, COUT), lambda b: (0, 0, 0)),
            pl.BlockSpec((1, COUT), lambda b: (0, 0)),
            pl.BlockSpec((1, COUT), lambda b: (0, 0)),
        ],
        out_specs=[pl.BlockSpec((1, N, COUT), lambda b: (b, 0, 0))],
        out_shape=[jax.ShapeDtypeStruct((nb, N, COUT), jnp.float32)],
    )


def kernel(x, W_conv, b_conv, bn_w, bn_b):
    del b_conv
    xr = x.reshape(B, C, N)
    w1 = W_conv[:, :C]
    w2 = W_conv[:, C:]
    wd = (w1 - w2).T
    w2t = w2.T
    hb = B // 2
    hn = hb * N
    bnw = bn_w.reshape(1, COUT)
    bnb = bn_b.reshape(1, COUT)
    halves = []
    for h in range(2):
        xh = xr[h * hb:(h + 1) * hb]
        idx, a, bm, stat = _prep(hb)(xh, wd, w2t)
        idx_flat = idx[:, :, :K].reshape(hn * K)
        m, mn = _sc_gather(hn)(bm.reshape(hn, COUT), idx_flat)
        halves.append((a, m, mn, stat))
    stat_all = jnp.concatenate([halves[0][3], halves[1][3]], axis=0)
    outs = []
    for h in range(2):
        a, m, mn, _ = halves[h]
        (o,) = _finalize(hb)(a, m.reshape(hb, N, COUT),
                             mn.reshape(hb, N, COUT), stat_all, bnw, bnb)
        outs.append(o)
    out = jnp.concatenate(outs, axis=0)
    return out.transpose(0, 2, 1).reshape(B, COUT, H, W)


# speedup vs baseline: 1.1256x; 1.0211x over previous
import functools

import jax
import jax.numpy as jnp
from jax import lax
from jax.experimental import pallas as pl
from jax.experimental.pallas import tpu as pltpu
from jax.experimental.pallas import tpu_sc as plsc

B, C, H, W, COUT, K = 8, 384, 24, 24, 768, 9
N = H * W
BN = B * N
NKB = float(B * N * K)

NC, NS = 2, 16
NW = NC * NS
PW = BN // NW
CH = 8
NCH = PW // CH
ROWS = CH * K
LG = COUT // 16


def _prep_body(x_ref, wd_ref, w2_ref, idx_ref, a_ref, bm_ref, stat_ref):
    b = pl.program_id(0)
    x = x_ref[0]
    nsq = jnp.sum(x * x, axis=0, keepdims=True)
    xn = x / jnp.maximum(jnp.sqrt(nsq), 1e-12)
    sqv = jnp.sum(xn * xn, axis=0, keepdims=True)
    gram = lax.dot_general(xn, xn, (((0,), (0,)), ((), ())),
                           preferred_element_type=jnp.float32)
    rank = sqv - 2.0 * gram
    iota_f = lax.broadcasted_iota(jnp.int32, (N, N), 1).astype(jnp.float32)
    rows = []
    d = rank
    e = jnp.zeros((N, N), jnp.float32)
    for _ in range(K):
        m = jnp.min(d, axis=1, keepdims=True)
        cand = jnp.where(d == m, iota_f, jnp.float32(N))
        am = jnp.min(cand, axis=1, keepdims=True)
        rows.append(am.astype(jnp.int32))
        hit = iota_f == am
        d = jnp.where(hit, jnp.inf, d)
        e = e + jnp.where(hit, 1.0, 0.0)
    idx_mat = jnp.concatenate(rows, axis=1) + b * N
    pad = jnp.zeros((N, 16 - K), jnp.int32) + b * N
    idx_ref[0] = jnp.concatenate([idx_mat, pad], axis=1)

    a = lax.dot_general(x, wd_ref[...], (((0,), (0,)), ((), ())),
                        preferred_element_type=jnp.float32)
    bm = lax.dot_general(x, w2_ref[...], (((0,), (0,)), ((), ())),
                         preferred_element_type=jnp.float32)
    a_ref[0] = a
    bm_ref[0] = bm
    s = lax.dot_general(e, bm, (((1,), (0,)), ((), ())),
                        preferred_element_type=jnp.float32)
    cnt = jnp.sum(e, axis=0, keepdims=True)
    bg2 = lax.dot_general(cnt, bm * bm, (((1,), (0,)), ((), ())),
                          preferred_element_type=jnp.float32)
    stat_ref[0] = jnp.concatenate([
        jnp.sum(a, axis=0, keepdims=True),
        jnp.sum(a * a, axis=0, keepdims=True),
        jnp.sum(s, axis=0, keepdims=True),
        bg2,
        jnp.sum(a * s, axis=0, keepdims=True),
    ], axis=0)


@functools.cache
def _prep(nb):
    return pl.pallas_call(
        _prep_body,
        grid=(nb,),
        in_specs=[
            pl.BlockSpec((1, C, N), lambda b: (b, 0, 0)),
            pl.BlockSpec((C, COUT), lambda b: (0, 0)),
            pl.BlockSpec((C, COUT), lambda b: (0, 0)),
        ],
        out_specs=[
            pl.BlockSpec((1, N, 16), lambda b: (b, 0, 0)),
            pl.BlockSpec((1, N, COUT), lambda b: (b, 0, 0)),
            pl.BlockSpec((1, N, COUT), lambda b: (b, 0, 0)),
            pl.BlockSpec((1, 5, COUT), lambda b: (b, 0, 0)),
        ],
        out_shape=[
            jax.ShapeDtypeStruct((nb, N, 16), jnp.int32),
            jax.ShapeDtypeStruct((nb, N, COUT), jnp.float32),
            jax.ShapeDtypeStruct((nb, N, COUT), jnp.float32),
            jax.ShapeDtypeStruct((nb, 5, COUT), jnp.float32),
        ],
    )


def _sc_body(pw, bm_hbm, idx_hbm, m_hbm, mn_hbm,
             idx_v, rows0, rows1, m_v, mn_v, gsem0, gsem1, ssem):
    nch = pw // CH
    cid = lax.axis_index("c")
    sid = lax.axis_index("s")
    wid = sid * NC + cid
    base_pt = wid * pw

    pltpu.sync_copy(idx_hbm.at[pl.ds(base_pt * K, pw * K)], idx_v)

    rows_bufs = (rows0, rows1)
    gsems = (gsem0, gsem1)

    def _start_gather(t):
        buf = rows_bufs[t % 2]
        return pltpu.async_copy(
            bm_hbm.at[idx_v.at[pl.ds(t * ROWS, ROWS)]], buf, gsems[t % 2])

    def _compute(t):
        rows_v = rows_bufs[t % 2]

        def _cg_body(cg, _):
            col = pl.ds(cg * 16, 16)
            for p in range(CH):
                r = rows_v[p * K, col]
                mx = r
                mn = r
                for k in range(1, K):
                    v = rows_v[p * K + k, col]
                    mx = jnp.maximum(mx, v)
                    mn = jnp.minimum(mn, v)
                m_v[p, col] = mx
                mn_v[p, col] = mn
            return 0

        lax.fori_loop(0, LG, _cg_body, 0)

    pending = _start_gather(0)
    store_m = None
    for t in range(nch):
        nxt = _start_gather(t + 1) if t + 1 < nch else None
        pending.wait()
        if store_m is not None:
            store_m.wait()
            store_mn.wait()
        _compute(t)
        p0 = base_pt + t * CH
        store_m = pltpu.async_copy(m_v, m_hbm.at[pl.ds(p0, CH)], ssem)
        store_mn = pltpu.async_copy(mn_v, mn_hbm.at[pl.ds(p0, CH)], ssem)
        pending = nxt
    store_m.wait()
    store_mn.wait()


@functools.cache
def _sc_gather(npts):
    pw = npts // NW
    return pl.kernel(
        functools.partial(_sc_body, pw),
        out_type=[
            jax.ShapeDtypeStruct((npts, COUT), jnp.float32),
            jax.ShapeDtypeStruct((npts, COUT), jnp.float32),
        ],
        mesh=plsc.VectorSubcoreMesh(core_axis_name="c", subcore_axis_name="s",
                                    num_cores=NC, num_subcores=NS),
        scratch_types=[
            pltpu.VMEM((pw * K,), jnp.int32),
            pltpu.VMEM((ROWS, COUT), jnp.float32),
            pltpu.VMEM((ROWS, COUT), jnp.float32),
            pltpu.VMEM((CH, COUT), jnp.float32),
            pltpu.VMEM((CH, COUT), jnp.float32),
            pltpu.SemaphoreType.DMA,
            pltpu.SemaphoreType.DMA,
            pltpu.SemaphoreType.DMA,
        ],
    )


def _gelu(y):
    return 0.5 * y * (1.0 + lax.erf(y * 0.7071067811865476))


def _fin_body(a_ref, m_ref, mn_ref, stat_ref, bnw_ref, bnb_ref, o_ref):
    sum_a = jnp.sum(stat_ref[:, 0, :], axis=0, keepdims=True)
    sum_a2 = jnp.sum(stat_ref[:, 1, :], axis=0, keepdims=True)
    sum_bg = jnp.sum(stat_ref[:, 2, :], axis=0, keepdims=True)
    sum_bg2 = jnp.sum(stat_ref[:, 3, :], axis=0, keepdims=True)
    sum_cr = jnp.sum(stat_ref[:, 4, :], axis=0, keepdims=True)
    mean = (K * sum_a + sum_bg) / NKB
    ex2 = (K * sum_a2 + 2.0 * sum_cr + sum_bg2) / NKB
    var = ex2 - mean * mean
    inv = bnw_ref[...] * lax.rsqrt(var + 1e-5)
    a = a_ref[0]
    y1 = (a + m_ref[0] - mean) * inv + bnb_ref[...]
    y2 = (a + mn_ref[0] - mean) * inv + bnb_ref[...]
    o_ref[0] = jnp.maximum(_gelu(y1), _gelu(y2))


@functools.cache
def _finalize(nb):
    return pl.pallas_call(
        _fin_body,
        grid=(nb,),
        in_specs=[
            pl.BlockSpec((1, N, COUT), lambda b: (b, 0, 0)),
            pl.BlockSpec((1, N, COUT), lambda b: (b, 0, 0)),
            pl.BlockSpec((1, N, COUT), lambda b: (b, 0, 0)),
            pl.BlockSpec((B, 5, COUT), lambda b: (0, 0, 0)),
            pl.BlockSpec((1, COUT), lambda b: (0, 0)),
            pl.BlockSpec((1, COUT), lambda b: (0, 0)),
        ],
        out_specs=[pl.BlockSpec((1, N, COUT), lambda b: (b, 0, 0))],
        out_shape=[jax.ShapeDtypeStruct((nb, N, COUT), jnp.float32)],
    )


def kernel(x, W_conv, b_conv, bn_w, bn_b):
    del b_conv
    xr = x.reshape(B, C, N)
    w1 = W_conv[:, :C]
    w2 = W_conv[:, C:]
    wd = (w1 - w2).T
    w2t = w2.T
    hb = B // 2
    hn = hb * N
    bnw = bn_w.reshape(1, COUT)
    bnb = bn_b.reshape(1, COUT)
    halves = []
    for h in range(2):
        xh = xr[h * hb:(h + 1) * hb]
        idx, a, bm, stat = _prep(hb)(xh, wd, w2t)
        idx_flat = idx[:, :, :K].reshape(hn * K)
        m, mn = _sc_gather(hn)(bm.reshape(hn, COUT), idx_flat)
        halves.append((a, m, mn, stat))
    stat_all = jnp.concatenate([halves[0][3], halves[1][3]], axis=0)
    outs = []
    for h in range(2):
        a, m, mn, _ = halves[h]
        (o,) = _finalize(hb)(a, m.reshape(hb, N, COUT),
                             mn.reshape(hb, N, COUT), stat_all, bnw, bnb)
        outs.append(o)
    out = jnp.concatenate(outs, axis=0)
    return out.transpose(0, 2, 1).reshape(B, COUT, H, W)
